# baseline (speedup 1.0000x reference)
import jax
import jax.numpy as jnp
import numpy as np
from jax.experimental import pallas as pl
from jax.experimental.pallas import tpu as pltpu

_B, _L, _D, _H, _NL, _V, _W = 4, 2048, 512, 8, 4, 258, 8
_HD = _D // _H
_SCALE_LOG2E = (1.0 / float(np.sqrt(_HD))) * 1.4426950408889634
_INV_LN2 = 1.4426950408889634
_TQ = 2048
_TP = 512


def _contract_last(a, b):
    return jax.lax.dot_general(a, b, (((1,), (1,)), ((), ())),
                               preferred_element_type=jnp.float32)


def _layernorm(h, s, b):
    mu = jnp.mean(h, axis=1, keepdims=True)
    d = h - mu
    var = jnp.mean(d * d, axis=1, keepdims=True)
    return d * jax.lax.rsqrt(var + 1e-5) * s + b


def _ln_qkv(h, s, b, w, qb):
    g = _layernorm(h, s, b)
    return (_contract_last(g.astype(jnp.bfloat16), w)
            + qb).astype(jnp.bfloat16)


def _entropy_kernel(x_ref, ent_ref):
    x = x_ref[...]
    fill_i = jnp.full((_B, 1), -1, jnp.int32)
    eqs = []
    for d in range(1, _W):
        xd = jnp.concatenate(
            [x[:, d:], jnp.broadcast_to(fill_i, (_B, d))], axis=1)
        eqs.append((x == xd).astype(jnp.float32))
    zf = jnp.zeros((_B, 1), jnp.float32)
    bks = []
    for a in range(1, _W):
        bks.append(jnp.concatenate(
            [jnp.broadcast_to(zf, (_B, a)), eqs[a - 1][:, :_L - a]], axis=1))
    valid = (x < 256).astype(jnp.float32)
    ssum = jnp.zeros((_B, _L), jnp.float32)
    nsum = jnp.zeros((_B, _L), jnp.float32)
    for j in range(_W):
        c = jnp.ones((_B, _L), jnp.float32)
        for a in range(1, j + 1):
            c = c + bks[a - 1]
        for d in range(1, _W - j):
            c = c + eqs[d - 1]
        g = valid * (jnp.log(c) * _INV_LN2)
        if j:
            g = jnp.concatenate(
                [g[:, j:], jnp.broadcast_to(zf, (_B, j))], axis=1)
            vj = jnp.concatenate(
                [valid[:, j:], jnp.broadcast_to(zf, (_B, j))], axis=1)
        else:
            vj = valid
        ssum = ssum + g
        nsum = nsum + vj
    ent = 0.375 * nsum - 0.125 * ssum
    lane = jax.lax.broadcasted_iota(jnp.int32, (_B, _L), 1)
    ent_ref[...] = jnp.where(lane <= _L - _W, ent, 0.0)


def _embed_kernel(xc_ref, ent_ref, emb_ref, entw_ref, entb_ref,
                  s1_ref, b1_ref, qw_ref, qb_ref, h_ref, qkv_ref):
    xs = xc_ref[0]
    iota = jax.lax.broadcasted_iota(jnp.int32, (_L, _V), 1)
    oh = (iota == xs).astype(jnp.float32)
    tok = jnp.dot(oh, emb_ref[...], preferred_element_type=jnp.float32)
    entc = ent_ref[0]
    h0 = tok + entc * entw_ref[...] + entb_ref[...]
    h_ref[0] = h0
    qkv_ref[0] = _ln_qkv(h0, s1_ref[0], b1_ref[0], qw_ref[0], qb_ref[0])


def _attn_kernel(q_ref, k_ref, v_ref, o_ref):
    k4 = k_ref[0]
    v4 = v_ref[0]

    def body(i, carry):
        base = pl.multiple_of(i * _TQ, _TQ)
        q4 = q_ref[0, pl.ds(base, _TQ), :]
        for hh in range(4):
            sl = slice(hh * _HD, (hh + 1) * _HD)
            sT = _contract_last(k4[:, sl], q4[:, sl])
            pT = jnp.exp2(sT * _SCALE_LOG2E)
            den = jnp.sum(pT, axis=0, keepdims=True)
            oT = jax.lax.dot_general(
                v4[:, sl], pT.astype(jnp.bfloat16),
                (((0,), (0,)), ((), ())),
                preferred_element_type=jnp.float32)
            o_ref[0, hh * _HD:(hh + 1) * _HD, pl.ds(base, _TQ)] = (
                (oT / den).astype(jnp.bfloat16))
        return carry

    jax.lax.fori_loop(0, _L // _TQ, body, 0)


def _ffn_block(h_ref, o_ref, aow_ref, aob_ref, s2_ref, b2_ref,
               f1w_ref, f1b_ref, f2w_ref, f2b_ref):
    ao = jax.lax.dot_general(o_ref[0], aow_ref[0], (((0,), (1,)), ((), ())),
                             preferred_element_type=jnp.float32)
    h1 = h_ref[0] + ao + aob_ref[0]
    g = _layernorm(h1, s2_ref[0], b2_ref[0])
    u = _contract_last(g.astype(jnp.bfloat16), f1w_ref[0]) + f1b_ref[0]
    u = 0.5 * u * (1.0 + jax.lax.erf(u * 0.7071067811865476))
    y = _contract_last(u.astype(jnp.bfloat16), f2w_ref[0]) + f2b_ref[0]
    return h1 + y


def _post_qkv_kernel(h_ref, o_ref, aow_ref, aob_ref, s2_ref, b2_ref,
                     f1w_ref, f1b_ref, f2w_ref, f2b_ref,
                     s1_ref, b1_ref, qw_ref, qb_ref, hout_ref, qkv_ref):
    h2 = _ffn_block(h_ref, o_ref, aow_ref, aob_ref, s2_ref, b2_ref,
                    f1w_ref, f1b_ref, f2w_ref, f2b_ref)
    hout_ref[0] = h2
    qkv_ref[0] = _ln_qkv(h2, s1_ref[0], b1_ref[0], qw_ref[0], qb_ref[0])


def _post_logits_kernel(h_ref, o_ref, aow_ref, aob_ref, s2_ref, b2_ref,
                        f1w_ref, f1b_ref, f2w_ref, f2b_ref,
                        ow_ref, logits_ref):
    h2 = _ffn_block(h_ref, o_ref, aow_ref, aob_ref, s2_ref, b2_ref,
                    f1w_ref, f1b_ref, f2w_ref, f2b_ref)
    logits_ref[0] = _contract_last(h2.astype(jnp.bfloat16), ow_ref[...])


def _cp(*sem):
    return pltpu.CompilerParams(dimension_semantics=sem,
                                vmem_limit_bytes=60 * 1024 * 1024)


def kernel(emb, ent_w, ent_b, qkv_w, qkv_b, ao_w, ao_b, ln1_s, ln1_b,
           ln2_s, ln2_b, ff1_w, ff1_b, ff2_w, ff2_b, out_w, x, patch_lengths):
    del patch_lengths

    ent = pl.pallas_call(
        _entropy_kernel,
        out_shape=jax.ShapeDtypeStruct((_B, _L), jnp.float32),
    )(x)

    xc = x.reshape(_B, _L, 1)
    entc = ent.reshape(_B, _L, 1)
    entw_row = ent_w.reshape(1, _D)
    entb_row = ent_b.reshape(1, _D)
    ln1_s3 = ln1_s.reshape(_NL, 1, _D)
    ln1_b3 = ln1_b.reshape(_NL, 1, _D)
    qkv_b3 = qkv_b.reshape(_NL, 1, 3 * _D)
    ao_b3 = ao_b.reshape(_NL, 1, _D)
    ln2_s3 = ln2_s.reshape(_NL, 1, _D)
    ln2_b3 = ln2_b.reshape(_NL, 1, _D)
    ff1_b3 = ff1_b.reshape(_NL, 1, 4 * _D)
    ff2_b3 = ff2_b.reshape(_NL, 1, _D)
    qkv_wb = qkv_w.astype(jnp.bfloat16)
    ao_wb = ao_w.astype(jnp.bfloat16)
    ff1_wb = ff1_w.astype(jnp.bfloat16)
    ff2_wb = ff2_w.astype(jnp.bfloat16)
    out_wb = out_w.astype(jnp.bfloat16)

    h, qkv = pl.pallas_call(
        _embed_kernel,
        grid=(_B,),
        in_specs=[
            pl.BlockSpec((1, _L, 1), lambda b: (b, 0, 0)),
            pl.BlockSpec((1, _L, 1), lambda b: (b, 0, 0)),
            pl.BlockSpec((_V, _D), lambda b: (0, 0)),
            pl.BlockSpec((1, _D), lambda b: (0, 0)),
            pl.BlockSpec((1, _D), lambda b: (0, 0)),
            pl.BlockSpec((1, 1, _D), lambda b: (0, 0, 0)),
            pl.BlockSpec((1, 1, _D), lambda b: (0, 0, 0)),
            pl.BlockSpec((1, 3 * _D, _D), lambda b: (0, 0, 0)),
            pl.BlockSpec((1, 1, 3 * _D), lambda b: (0, 0, 0)),
        ],
        out_specs=(
            pl.BlockSpec((1, _L, _D), lambda b: (b, 0, 0)),
            pl.BlockSpec((1, _L, 3 * _D), lambda b: (b, 0, 0)),
        ),
        out_shape=(
            jax.ShapeDtypeStruct((_B, _L, _D), jnp.float32),
            jax.ShapeDtypeStruct((_B, _L, 3 * _D), jnp.bfloat16),
        ),
        compiler_params=_cp("parallel"),
    )(xc, entc, emb, entw_row, entb_row, ln1_s3, ln1_b3, qkv_wb, qkv_b3)

    logits = None
    for li in range(_NL):
        o = pl.pallas_call(
            _attn_kernel,
            grid=(_B * _H // 4,),
            in_specs=[
                pl.BlockSpec((1, _L, 256), lambda p: (p // 2, 0, p % 2)),
                pl.BlockSpec((1, _L, 256), lambda p: (p // 2, 0, 2 + p % 2)),
                pl.BlockSpec((1, _L, 256), lambda p: (p // 2, 0, 4 + p % 2)),
            ],
            out_specs=pl.BlockSpec((1, 256, _L), lambda p: (p // 2, p % 2, 0)),
            out_shape=jax.ShapeDtypeStruct((_B, _D, _L), jnp.bfloat16),
            compiler_params=_cp("parallel"),
        )(qkv, qkv, qkv)

        post_specs = [
            pl.BlockSpec((1, _TP, _D), lambda b, t: (b, t, 0)),
            pl.BlockSpec((1, _D, _TP), lambda b, t: (b, 0, t)),
            pl.BlockSpec((1, _D, _D), lambda b, t, li=li: (li, 0, 0)),
            pl.BlockSpec((1, 1, _D), lambda b, t, li=li: (li, 0, 0)),
            pl.BlockSpec((1, 1, _D), lambda b, t, li=li: (li, 0, 0)),
            pl.BlockSpec((1, 1, _D), lambda b, t, li=li: (li, 0, 0)),
            pl.BlockSpec((1, 4 * _D, _D), lambda b, t, li=li: (li, 0, 0)),
            pl.BlockSpec((1, 1, 4 * _D), lambda b, t, li=li: (li, 0, 0)),
            pl.BlockSpec((1, _D, 4 * _D), lambda b, t, li=li: (li, 0, 0)),
            pl.BlockSpec((1, 1, _D), lambda b, t, li=li: (li, 0, 0)),
        ]
        if li < _NL - 1:
            nxt = li + 1
            h, qkv = pl.pallas_call(
                _post_qkv_kernel,
                grid=(_B, _L // _TP),
                in_specs=post_specs + [
                    pl.BlockSpec((1, 1, _D), lambda b, t, li=nxt: (li, 0, 0)),
                    pl.BlockSpec((1, 1, _D), lambda b, t, li=nxt: (li, 0, 0)),
                    pl.BlockSpec((1, 3 * _D, _D),
                                 lambda b, t, li=nxt: (li, 0, 0)),
                    pl.BlockSpec((1, 1, 3 * _D),
                                 lambda b, t, li=nxt: (li, 0, 0)),
                ],
                out_specs=(
                    pl.BlockSpec((1, _TP, _D), lambda b, t: (b, t, 0)),
                    pl.BlockSpec((1, _TP, 3 * _D), lambda b, t: (b, t, 0)),
                ),
                out_shape=(
                    jax.ShapeDtypeStruct((_B, _L, _D), jnp.float32),
                    jax.ShapeDtypeStruct((_B, _L, 3 * _D), jnp.bfloat16),
                ),
                compiler_params=_cp("parallel", "arbitrary"),
            )(h, o, ao_wb, ao_b3, ln2_s3, ln2_b3, ff1_wb, ff1_b3,
              ff2_wb, ff2_b3, ln1_s3, ln1_b3, qkv_wb, qkv_b3)
        else:
            logits = pl.pallas_call(
                _post_logits_kernel,
                grid=(_B, _L // _TP),
                in_specs=post_specs + [
                    pl.BlockSpec((_V, _D), lambda b, t: (0, 0)),
                ],
                out_specs=pl.BlockSpec((1, _TP, _V), lambda b, t: (b, t, 0)),
                out_shape=jax.ShapeDtypeStruct((_B, _L, _V), jnp.float32),
                compiler_params=_cp("parallel", "arbitrary"),
            )(h, o, ao_wb, ao_b3, ln2_s3, ln2_b3, ff1_wb, ff1_b3,
              ff2_wb, ff2_b3, out_wb)
    return logits


# speedup vs baseline: 1.0307x; 1.0307x over previous
import jax
import jax.numpy as jnp
import numpy as np
from jax.experimental import pallas as pl
from jax.experimental.pallas import tpu as pltpu

_B, _L, _D, _H, _NL, _V, _W = 4, 2048, 512, 8, 4, 258, 8
_HD = _D // _H
_SCALE_LOG2E = (1.0 / float(np.sqrt(_HD))) * 1.4426950408889634
_INV_LN2 = 1.4426950408889634
_TQ = 2048
_TP = 1024


def _contract_last(a, b):
    return jax.lax.dot_general(a, b, (((1,), (1,)), ((), ())),
                               preferred_element_type=jnp.float32)


def _layernorm(h, s, b):
    mu = jnp.mean(h, axis=1, keepdims=True)
    d = h - mu
    var = jnp.mean(d * d, axis=1, keepdims=True)
    return d * jax.lax.rsqrt(var + 1e-5) * s + b


def _ln_qkv(h, s, b, w, qb):
    g = _layernorm(h, s, b)
    return (_contract_last(g.astype(jnp.bfloat16), w)
            + qb).astype(jnp.bfloat16)


def _entropy_kernel(x_ref, ent_ref):
    x = x_ref[...]
    fill_i = jnp.full((_B, 1), -1, jnp.int32)
    eqs = []
    for d in range(1, _W):
        xd = jnp.concatenate(
            [x[:, d:], jnp.broadcast_to(fill_i, (_B, d))], axis=1)
        eqs.append((x == xd).astype(jnp.float32))
    zf = jnp.zeros((_B, 1), jnp.float32)
    bks = []
    for a in range(1, _W):
        bks.append(jnp.concatenate(
            [jnp.broadcast_to(zf, (_B, a)), eqs[a - 1][:, :_L - a]], axis=1))
    valid = (x < 256).astype(jnp.float32)
    ssum = jnp.zeros((_B, _L), jnp.float32)
    nsum = jnp.zeros((_B, _L), jnp.float32)
    for j in range(_W):
        c = jnp.ones((_B, _L), jnp.float32)
        for a in range(1, j + 1):
            c = c + bks[a - 1]
        for d in range(1, _W - j):
            c = c + eqs[d - 1]
        g = valid * (jnp.log(c) * _INV_LN2)
        if j:
            g = jnp.concatenate(
                [g[:, j:], jnp.broadcast_to(zf, (_B, j))], axis=1)
            vj = jnp.concatenate(
                [valid[:, j:], jnp.broadcast_to(zf, (_B, j))], axis=1)
        else:
            vj = valid
        ssum = ssum + g
        nsum = nsum + vj
    ent = 0.375 * nsum - 0.125 * ssum
    lane = jax.lax.broadcasted_iota(jnp.int32, (_B, _L), 1)
    ent_ref[...] = jnp.where(lane <= _L - _W, ent, 0.0)


def _embed_kernel(xc_ref, ent_ref, emb_ref, entw_ref, entb_ref,
                  s1_ref, b1_ref, qw_ref, qb_ref, h_ref, qkv_ref):
    xs = xc_ref[0]
    iota = jax.lax.broadcasted_iota(jnp.int32, (_L, _V), 1)
    oh = (iota == xs).astype(jnp.float32)
    tok = jnp.dot(oh, emb_ref[...], preferred_element_type=jnp.float32)
    entc = ent_ref[0]
    h0 = tok + entc * entw_ref[...] + entb_ref[...]
    h_ref[0] = h0
    qkv_ref[0] = _ln_qkv(h0, s1_ref[0], b1_ref[0], qw_ref[0], qb_ref[0])


def _attn_kernel(q_ref, k_ref, v_ref, o_ref):
    k4 = k_ref[0]
    v4 = v_ref[0]

    def body(i, carry):
        base = pl.multiple_of(i * _TQ, _TQ)
        q4 = q_ref[0, pl.ds(base, _TQ), :]
        for hh in range(4):
            sl = slice(hh * _HD, (hh + 1) * _HD)
            sT = _contract_last(k4[:, sl], q4[:, sl])
            pT = jnp.exp2(sT * _SCALE_LOG2E)
            den = jnp.sum(pT, axis=0, keepdims=True)
            oT = jax.lax.dot_general(
                v4[:, sl], pT.astype(jnp.bfloat16),
                (((0,), (0,)), ((), ())),
                preferred_element_type=jnp.float32)
            o_ref[0, hh * _HD:(hh + 1) * _HD, pl.ds(base, _TQ)] = (
                (oT / den).astype(jnp.bfloat16))
        return carry

    jax.lax.fori_loop(0, _L // _TQ, body, 0)


def _ffn_block(h_ref, o_ref, aow_ref, aob_ref, s2_ref, b2_ref,
               f1w_ref, f1b_ref, f2w_ref, f2b_ref):
    ao = jax.lax.dot_general(o_ref[0], aow_ref[0], (((0,), (1,)), ((), ())),
                             preferred_element_type=jnp.float32)
    h1 = h_ref[0] + ao + aob_ref[0]
    g = _layernorm(h1, s2_ref[0], b2_ref[0])
    g16 = g.astype(jnp.bfloat16)
    y = f2b_ref[0]
    for nn in range(2):
        csl = slice(nn * 2 * _D, (nn + 1) * 2 * _D)
        u = _contract_last(g16, f1w_ref[0, csl, :]) + f1b_ref[0, :, csl]
        u = 0.5 * u * (1.0 + jax.lax.erf(u * 0.7071067811865476))
        y = y + _contract_last(u.astype(jnp.bfloat16), f2w_ref[0, :, csl])
    return h1 + y


def _post_qkv_kernel(h_ref, o_ref, aow_ref, aob_ref, s2_ref, b2_ref,
                     f1w_ref, f1b_ref, f2w_ref, f2b_ref,
                     s1_ref, b1_ref, qw_ref, qb_ref, hout_ref, qkv_ref):
    h2 = _ffn_block(h_ref, o_ref, aow_ref, aob_ref, s2_ref, b2_ref,
                    f1w_ref, f1b_ref, f2w_ref, f2b_ref)
    hout_ref[0] = h2
    qkv_ref[0] = _ln_qkv(h2, s1_ref[0], b1_ref[0], qw_ref[0], qb_ref[0])


def _post_logits_kernel(h_ref, o_ref, aow_ref, aob_ref, s2_ref, b2_ref,
                        f1w_ref, f1b_ref, f2w_ref, f2b_ref,
                        ow_ref, logits_ref):
    h2 = _ffn_block(h_ref, o_ref, aow_ref, aob_ref, s2_ref, b2_ref,
                    f1w_ref, f1b_ref, f2w_ref, f2b_ref)
    logits_ref[0] = _contract_last(h2.astype(jnp.bfloat16), ow_ref[...])


def _cp(*sem):
    return pltpu.CompilerParams(dimension_semantics=sem,
                                vmem_limit_bytes=60 * 1024 * 1024)


def kernel(emb, ent_w, ent_b, qkv_w, qkv_b, ao_w, ao_b, ln1_s, ln1_b,
           ln2_s, ln2_b, ff1_w, ff1_b, ff2_w, ff2_b, out_w, x, patch_lengths):
    del patch_lengths

    ent = pl.pallas_call(
        _entropy_kernel,
        out_shape=jax.ShapeDtypeStruct((_B, _L), jnp.float32),
    )(x)

    xc = x.reshape(_B, _L, 1)
    entc = ent.reshape(_B, _L, 1)
    entw_row = ent_w.reshape(1, _D)
    entb_row = ent_b.reshape(1, _D)
    ln1_s3 = ln1_s.reshape(_NL, 1, _D)
    ln1_b3 = ln1_b.reshape(_NL, 1, _D)
    qkv_b3 = qkv_b.reshape(_NL, 1, 3 * _D)
    ao_b3 = ao_b.reshape(_NL, 1, _D)
    ln2_s3 = ln2_s.reshape(_NL, 1, _D)
    ln2_b3 = ln2_b.reshape(_NL, 1, _D)
    ff1_b3 = ff1_b.reshape(_NL, 1, 4 * _D)
    ff2_b3 = ff2_b.reshape(_NL, 1, _D)
    qkv_wb = qkv_w.astype(jnp.bfloat16)
    ao_wb = ao_w.astype(jnp.bfloat16)
    ff1_wb = ff1_w.astype(jnp.bfloat16)
    ff2_wb = ff2_w.astype(jnp.bfloat16)
    out_wb = out_w.astype(jnp.bfloat16)

    h, qkv = pl.pallas_call(
        _embed_kernel,
        grid=(_B,),
        in_specs=[
            pl.BlockSpec((1, _L, 1), lambda b: (b, 0, 0)),
            pl.BlockSpec((1, _L, 1), lambda b: (b, 0, 0)),
            pl.BlockSpec((_V, _D), lambda b: (0, 0)),
            pl.BlockSpec((1, _D), lambda b: (0, 0)),
            pl.BlockSpec((1, _D), lambda b: (0, 0)),
            pl.BlockSpec((1, 1, _D), lambda b: (0, 0, 0)),
            pl.BlockSpec((1, 1, _D), lambda b: (0, 0, 0)),
            pl.BlockSpec((1, 3 * _D, _D), lambda b: (0, 0, 0)),
            pl.BlockSpec((1, 1, 3 * _D), lambda b: (0, 0, 0)),
        ],
        out_specs=(
            pl.BlockSpec((1, _L, _D), lambda b: (b, 0, 0)),
            pl.BlockSpec((1, _L, 3 * _D), lambda b: (b, 0, 0)),
        ),
        out_shape=(
            jax.ShapeDtypeStruct((_B, _L, _D), jnp.float32),
            jax.ShapeDtypeStruct((_B, _L, 3 * _D), jnp.bfloat16),
        ),
        compiler_params=_cp("parallel"),
    )(xc, entc, emb, entw_row, entb_row, ln1_s3, ln1_b3, qkv_wb, qkv_b3)

    logits = None
    for li in range(_NL):
        o = pl.pallas_call(
            _attn_kernel,
            grid=(_B * _H // 4,),
            in_specs=[
                pl.BlockSpec((1, _L, 256), lambda p: (p // 2, 0, p % 2)),
                pl.BlockSpec((1, _L, 256), lambda p: (p // 2, 0, 2 + p % 2)),
                pl.BlockSpec((1, _L, 256), lambda p: (p // 2, 0, 4 + p % 2)),
            ],
            out_specs=pl.BlockSpec((1, 256, _L), lambda p: (p // 2, p % 2, 0)),
            out_shape=jax.ShapeDtypeStruct((_B, _D, _L), jnp.bfloat16),
            compiler_params=_cp("parallel"),
        )(qkv, qkv, qkv)

        post_specs = [
            pl.BlockSpec((1, _TP, _D), lambda b, t: (b, t, 0)),
            pl.BlockSpec((1, _D, _TP), lambda b, t: (b, 0, t)),
            pl.BlockSpec((1, _D, _D), lambda b, t, li=li: (li, 0, 0)),
            pl.BlockSpec((1, 1, _D), lambda b, t, li=li: (li, 0, 0)),
            pl.BlockSpec((1, 1, _D), lambda b, t, li=li: (li, 0, 0)),
            pl.BlockSpec((1, 1, _D), lambda b, t, li=li: (li, 0, 0)),
            pl.BlockSpec((1, 4 * _D, _D), lambda b, t, li=li: (li, 0, 0)),
            pl.BlockSpec((1, 1, 4 * _D), lambda b, t, li=li: (li, 0, 0)),
            pl.BlockSpec((1, _D, 4 * _D), lambda b, t, li=li: (li, 0, 0)),
            pl.BlockSpec((1, 1, _D), lambda b, t, li=li: (li, 0, 0)),
        ]
        if li < _NL - 1:
            nxt = li + 1
            h, qkv = pl.pallas_call(
                _post_qkv_kernel,
                grid=(_B, _L // _TP),
                in_specs=post_specs + [
                    pl.BlockSpec((1, 1, _D), lambda b, t, li=nxt: (li, 0, 0)),
                    pl.BlockSpec((1, 1, _D), lambda b, t, li=nxt: (li, 0, 0)),
                    pl.BlockSpec((1, 3 * _D, _D),
                                 lambda b, t, li=nxt: (li, 0, 0)),
                    pl.BlockSpec((1, 1, 3 * _D),
                                 lambda b, t, li=nxt: (li, 0, 0)),
                ],
                out_specs=(
                    pl.BlockSpec((1, _TP, _D), lambda b, t: (b, t, 0)),
                    pl.BlockSpec((1, _TP, 3 * _D), lambda b, t: (b, t, 0)),
                ),
                out_shape=(
                    jax.ShapeDtypeStruct((_B, _L, _D), jnp.float32),
                    jax.ShapeDtypeStruct((_B, _L, 3 * _D), jnp.bfloat16),
                ),
                compiler_params=_cp("parallel", "arbitrary"),
            )(h, o, ao_wb, ao_b3, ln2_s3, ln2_b3, ff1_wb, ff1_b3,
              ff2_wb, ff2_b3, ln1_s3, ln1_b3, qkv_wb, qkv_b3)
        else:
            logits = pl.pallas_call(
                _post_logits_kernel,
                grid=(_B, _L // _TP),
                in_specs=post_specs + [
                    pl.BlockSpec((_V, _D), lambda b, t: (0, 0)),
                ],
                out_specs=pl.BlockSpec((1, _TP, _V), lambda b, t: (b, t, 0)),
                out_shape=jax.ShapeDtypeStruct((_B, _L, _V), jnp.float32),
                compiler_params=_cp("parallel", "arbitrary"),
            )(h, o, ao_wb, ao_b3, ln2_s3, ln2_b3, ff1_wb, ff1_b3,
              ff2_wb, ff2_b3, out_wb)
    return logits


# speedup vs baseline: 1.0348x; 1.0040x over previous
import jax
import jax.numpy as jnp
import numpy as np
from jax.experimental import pallas as pl
from jax.experimental.pallas import tpu as pltpu

_B, _L, _D, _H, _NL, _V, _W = 4, 2048, 512, 8, 4, 258, 8
_HD = _D // _H
_SCALE_LOG2E = (1.0 / float(np.sqrt(_HD))) * 1.4426950408889634
_INV_LN2 = 1.4426950408889634
_TQ = 2048
_TP = 1024


def _contract_last(a, b):
    return jax.lax.dot_general(a, b, (((1,), (1,)), ((), ())),
                               preferred_element_type=jnp.float32)


def _layernorm(h, s, b):
    mu = jnp.mean(h, axis=1, keepdims=True)
    d = h - mu
    var = jnp.mean(d * d, axis=1, keepdims=True)
    return d * jax.lax.rsqrt(var + 1e-5) * s + b


def _ln_qkv(h, s, b, w, qb):
    g = _layernorm(h, s, b)
    return (_contract_last(g.astype(jnp.bfloat16), w)
            + qb).astype(jnp.bfloat16)


def _entropy_kernel(x_ref, ent_ref):
    x = x_ref[...]
    fill_i = jnp.full((_B, 1), -1, jnp.int32)
    eqs = []
    for d in range(1, _W):
        xd = jnp.concatenate(
            [x[:, d:], jnp.broadcast_to(fill_i, (_B, d))], axis=1)
        eqs.append((x == xd).astype(jnp.float32))
    zf = jnp.zeros((_B, 1), jnp.float32)
    bks = []
    for a in range(1, _W):
        bks.append(jnp.concatenate(
            [jnp.broadcast_to(zf, (_B, a)), eqs[a - 1][:, :_L - a]], axis=1))
    valid = (x < 256).astype(jnp.float32)
    ssum = jnp.zeros((_B, _L), jnp.float32)
    nsum = jnp.zeros((_B, _L), jnp.float32)
    for j in range(_W):
        c = jnp.ones((_B, _L), jnp.float32)
        for a in range(1, j + 1):
            c = c + bks[a - 1]
        for d in range(1, _W - j):
            c = c + eqs[d - 1]
        g = valid * (jnp.log(c) * _INV_LN2)
        if j:
            g = jnp.concatenate(
                [g[:, j:], jnp.broadcast_to(zf, (_B, j))], axis=1)
            vj = jnp.concatenate(
                [valid[:, j:], jnp.broadcast_to(zf, (_B, j))], axis=1)
        else:
            vj = valid
        ssum = ssum + g
        nsum = nsum + vj
    ent = 0.375 * nsum - 0.125 * ssum
    lane = jax.lax.broadcasted_iota(jnp.int32, (_B, _L), 1)
    ent_ref[...] = jnp.where(lane <= _L - _W, ent, 0.0)


def _embed_kernel(xc_ref, ent_ref, emb_ref, entw_ref, entb_ref,
                  s1_ref, b1_ref, qw_ref, qb_ref, h_ref, qkv_ref):
    xs = xc_ref[0]
    iota = jax.lax.broadcasted_iota(jnp.int32, (_L, _V), 1)
    oh = (iota == xs).astype(jnp.float32)
    tok = jnp.dot(oh, emb_ref[...], preferred_element_type=jnp.float32)
    entc = ent_ref[0]
    h0 = tok + entc * entw_ref[...] + entb_ref[...]
    h_ref[0] = h0
    qkv_ref[0] = _ln_qkv(h0, s1_ref[0], b1_ref[0], qw_ref[0], qb_ref[0])


def _attn_kernel(q_ref, k_ref, v_ref, o_ref):
    k4 = k_ref[0]
    v4 = v_ref[0]

    def body(i, carry):
        base = pl.multiple_of(i * _TQ, _TQ)
        q4 = q_ref[0, pl.ds(base, _TQ), :]
        for hh in range(4):
            sl = slice(hh * _HD, (hh + 1) * _HD)
            sT = _contract_last(k4[:, sl], q4[:, sl])
            pT = jnp.exp2(sT * _SCALE_LOG2E)
            den = jnp.sum(pT, axis=0, keepdims=True)
            oT = jax.lax.dot_general(
                v4[:, sl], pT.astype(jnp.bfloat16),
                (((0,), (0,)), ((), ())),
                preferred_element_type=jnp.float32)
            o_ref[0, hh * _HD:(hh + 1) * _HD, pl.ds(base, _TQ)] = (
                (oT / den).astype(jnp.bfloat16))
        return carry

    jax.lax.fori_loop(0, _L // _TQ, body, 0)


def _ffn_block(h_ref, o_ref, aow_ref, aob_ref, s2_ref, b2_ref,
               f1w_ref, f1b_ref, f2w_ref, f2b_ref):
    ao = jax.lax.dot_general(o_ref[0], aow_ref[0], (((0,), (1,)), ((), ())),
                             preferred_element_type=jnp.float32)
    h1 = h_ref[0] + ao + aob_ref[0]
    g = _layernorm(h1, s2_ref[0], b2_ref[0])
    u = _contract_last(g.astype(jnp.bfloat16), f1w_ref[0]) + f1b_ref[0]
    u = 0.5 * u * (1.0 + jax.lax.erf(u * 0.7071067811865476))
    y = _contract_last(u.astype(jnp.bfloat16), f2w_ref[0]) + f2b_ref[0]
    return h1 + y


def _post_qkv_kernel(h_ref, o_ref, aow_ref, aob_ref, s2_ref, b2_ref,
                     f1w_ref, f1b_ref, f2w_ref, f2b_ref,
                     s1_ref, b1_ref, qw_ref, qb_ref, hout_ref, qkv_ref):
    h2 = _ffn_block(h_ref, o_ref, aow_ref, aob_ref, s2_ref, b2_ref,
                    f1w_ref, f1b_ref, f2w_ref, f2b_ref)
    hout_ref[0] = h2
    qkv_ref[0] = _ln_qkv(h2, s1_ref[0], b1_ref[0], qw_ref[0], qb_ref[0])


def _post_logits_kernel(h_ref, o_ref, aow_ref, aob_ref, s2_ref, b2_ref,
                        f1w_ref, f1b_ref, f2w_ref, f2b_ref,
                        ow_ref, logits_ref):
    h2 = _ffn_block(h_ref, o_ref, aow_ref, aob_ref, s2_ref, b2_ref,
                    f1w_ref, f1b_ref, f2w_ref, f2b_ref)
    logits_ref[0] = _contract_last(h2.astype(jnp.bfloat16), ow_ref[...])


def _cp(*sem):
    return pltpu.CompilerParams(dimension_semantics=sem,
                                vmem_limit_bytes=60 * 1024 * 1024)


def kernel(emb, ent_w, ent_b, qkv_w, qkv_b, ao_w, ao_b, ln1_s, ln1_b,
           ln2_s, ln2_b, ff1_w, ff1_b, ff2_w, ff2_b, out_w, x, patch_lengths):
    del patch_lengths

    ent = pl.pallas_call(
        _entropy_kernel,
        out_shape=jax.ShapeDtypeStruct((_B, _L), jnp.float32),
    )(x)

    xc = x.reshape(_B, _L, 1)
    entc = ent.reshape(_B, _L, 1)
    entw_row = ent_w.reshape(1, _D)
    entb_row = ent_b.reshape(1, _D)
    ln1_s3 = ln1_s.reshape(_NL, 1, _D)
    ln1_b3 = ln1_b.reshape(_NL, 1, _D)
    qkv_b3 = qkv_b.reshape(_NL, 1, 3 * _D)
    ao_b3 = ao_b.reshape(_NL, 1, _D)
    ln2_s3 = ln2_s.reshape(_NL, 1, _D)
    ln2_b3 = ln2_b.reshape(_NL, 1, _D)
    ff1_b3 = ff1_b.reshape(_NL, 1, 4 * _D)
    ff2_b3 = ff2_b.reshape(_NL, 1, _D)
    qkv_wb = qkv_w.astype(jnp.bfloat16)
    ao_wb = ao_w.astype(jnp.bfloat16)
    ff1_wb = ff1_w.astype(jnp.bfloat16)
    ff2_wb = ff2_w.astype(jnp.bfloat16)
    out_wb = out_w.astype(jnp.bfloat16)

    h, qkv = pl.pallas_call(
        _embed_kernel,
        grid=(_B,),
        in_specs=[
            pl.BlockSpec((1, _L, 1), lambda b: (b, 0, 0)),
            pl.BlockSpec((1, _L, 1), lambda b: (b, 0, 0)),
            pl.BlockSpec((_V, _D), lambda b: (0, 0)),
            pl.BlockSpec((1, _D), lambda b: (0, 0)),
            pl.BlockSpec((1, _D), lambda b: (0, 0)),
            pl.BlockSpec((1, 1, _D), lambda b: (0, 0, 0)),
            pl.BlockSpec((1, 1, _D), lambda b: (0, 0, 0)),
            pl.BlockSpec((1, 3 * _D, _D), lambda b: (0, 0, 0)),
            pl.BlockSpec((1, 1, 3 * _D), lambda b: (0, 0, 0)),
        ],
        out_specs=(
            pl.BlockSpec((1, _L, _D), lambda b: (b, 0, 0)),
            pl.BlockSpec((1, _L, 3 * _D), lambda b: (b, 0, 0)),
        ),
        out_shape=(
            jax.ShapeDtypeStruct((_B, _L, _D), jnp.float32),
            jax.ShapeDtypeStruct((_B, _L, 3 * _D), jnp.bfloat16),
        ),
        compiler_params=_cp("parallel"),
    )(xc, entc, emb, entw_row, entb_row, ln1_s3, ln1_b3, qkv_wb, qkv_b3)

    logits = None
    for li in range(_NL):
        o = pl.pallas_call(
            _attn_kernel,
            grid=(_B * _H // 4,),
            in_specs=[
                pl.BlockSpec((1, _L, 256), lambda p: (p // 2, 0, p % 2)),
                pl.BlockSpec((1, _L, 256), lambda p: (p // 2, 0, 2 + p % 2)),
                pl.BlockSpec((1, _L, 256), lambda p: (p // 2, 0, 4 + p % 2)),
            ],
            out_specs=pl.BlockSpec((1, 256, _L), lambda p: (p // 2, p % 2, 0)),
            out_shape=jax.ShapeDtypeStruct((_B, _D, _L), jnp.bfloat16),
            compiler_params=_cp("parallel"),
        )(qkv, qkv, qkv)

        post_specs = [
            pl.BlockSpec((1, _TP, _D), lambda b, t: (b, t, 0)),
            pl.BlockSpec((1, _D, _TP), lambda b, t: (b, 0, t)),
            pl.BlockSpec((1, _D, _D), lambda b, t, li=li: (li, 0, 0)),
            pl.BlockSpec((1, 1, _D), lambda b, t, li=li: (li, 0, 0)),
            pl.BlockSpec((1, 1, _D), lambda b, t, li=li: (li, 0, 0)),
            pl.BlockSpec((1, 1, _D), lambda b, t, li=li: (li, 0, 0)),
            pl.BlockSpec((1, 4 * _D, _D), lambda b, t, li=li: (li, 0, 0)),
            pl.BlockSpec((1, 1, 4 * _D), lambda b, t, li=li: (li, 0, 0)),
            pl.BlockSpec((1, _D, 4 * _D), lambda b, t, li=li: (li, 0, 0)),
            pl.BlockSpec((1, 1, _D), lambda b, t, li=li: (li, 0, 0)),
        ]
        if li < _NL - 1:
            nxt = li + 1
            h, qkv = pl.pallas_call(
                _post_qkv_kernel,
                grid=(_B, _L // _TP),
                in_specs=post_specs + [
                    pl.BlockSpec((1, 1, _D), lambda b, t, li=nxt: (li, 0, 0)),
                    pl.BlockSpec((1, 1, _D), lambda b, t, li=nxt: (li, 0, 0)),
                    pl.BlockSpec((1, 3 * _D, _D),
                                 lambda b, t, li=nxt: (li, 0, 0)),
                    pl.BlockSpec((1, 1, 3 * _D),
                                 lambda b, t, li=nxt: (li, 0, 0)),
                ],
                out_specs=(
                    pl.BlockSpec((1, _TP, _D), lambda b, t: (b, t, 0)),
                    pl.BlockSpec((1, _TP, 3 * _D), lambda b, t: (b, t, 0)),
                ),
                out_shape=(
                    jax.ShapeDtypeStruct((_B, _L, _D), jnp.float32),
                    jax.ShapeDtypeStruct((_B, _L, 3 * _D), jnp.bfloat16),
                ),
                compiler_params=_cp("parallel", "arbitrary"),
            )(h, o, ao_wb, ao_b3, ln2_s3, ln2_b3, ff1_wb, ff1_b3,
              ff2_wb, ff2_b3, ln1_s3, ln1_b3, qkv_wb, qkv_b3)
        else:
            logits = pl.pallas_call(
                _post_logits_kernel,
                grid=(_B, _L // _TP),
                in_specs=post_specs + [
                    pl.BlockSpec((_V, _D), lambda b, t: (0, 0)),
                ],
                out_specs=pl.BlockSpec((1, _TP, _V), lambda b, t: (b, t, 0)),
                out_shape=jax.ShapeDtypeStruct((_B, _L, _V), jnp.float32),
                compiler_params=_cp("parallel", "arbitrary"),
            )(h, o, ao_wb, ao_b3, ln2_s3, ln2_b3, ff1_wb, ff1_b3,
              ff2_wb, ff2_b3, out_wb)
    return logits
